# baseline (speedup 1.0000x reference)
import functools

import jax
import jax.numpy as jnp
import numpy as np
from jax import lax
from jax.experimental import pallas as pl
from jax.experimental.pallas import tpu as pltpu
from jax.experimental.pallas import tpu_sc as plsc

LETTER_VOCAB = 26
EMBED = 128
NFB = 3
NCOMB = LETTER_VOCAB * NFB
ROW = EMBED + NFB

NC = 2
NS = 16
NW = NC * NS
BK = 128


def _table_body(hot_ref, tab_ref, fbh_ref, t_ref):
    lets = jnp.dot(hot_ref[...], tab_ref[...], preferred_element_type=jnp.float32)
    lane = lax.broadcasted_iota(jnp.int32, (NCOMB, ROW), 1)
    letp = jnp.pad(lets, ((0, 0), (0, NFB)))
    fbp = jnp.pad(fbh_ref[...], ((0, 0), (EMBED, 0)))
    t_ref[...] = jnp.where(lane < EMBED, letp, fbp)


def _build_table(letter_table):
    hot = jnp.asarray(np.repeat(np.eye(LETTER_VOCAB, dtype=np.float32), NFB, axis=0))
    fbh = jnp.asarray(np.tile(np.eye(NFB, dtype=np.float32), (LETTER_VOCAB, 1)))
    return pl.pallas_call(
        _table_body,
        out_shape=jax.ShapeDtypeStruct((NCOMB, ROW), jnp.float32),
    )(hot, letter_table, fbh)


def _meta_body(t_ref, r_ref, o_ref):
    lane = lax.broadcasted_iota(jnp.int32, o_ref.shape, 1)
    o_ref[...] = jnp.where(lane == 0, t_ref[...], r_ref[...])


def _build_meta(turn_number, remaining):
    b = turn_number.shape[0]
    blk = 2048
    return pl.pallas_call(
        _meta_body,
        grid=(b // blk,),
        in_specs=[
            pl.BlockSpec((blk, 1), lambda i: (i, 0)),
            pl.BlockSpec((blk, 1), lambda i: (i, 0)),
        ],
        out_specs=pl.BlockSpec((blk, 2), lambda i: (i, 0)),
        out_shape=jax.ShapeDtypeStruct((b, 2), jnp.float32),
    )(turn_number.reshape(b, 1), remaining.reshape(b, 1))


def _sc_gather(t_hbm, lflat, fflat):
    rows_total = lflat.shape[0]
    assert rows_total % (NW * BK) == 0
    per_w = rows_total // NW
    nblk = per_w // BK
    mesh = plsc.VectorSubcoreMesh(core_axis_name="c", subcore_axis_name="s")

    @functools.partial(
        pl.kernel,
        mesh=mesh,
        out_type=jax.ShapeDtypeStruct((rows_total, ROW), jnp.float32),
        compiler_params=pltpu.CompilerParams(use_tc_tiling_on_sc=False),
        scratch_types=[
            pltpu.VMEM((BK,), jnp.int32),
            pltpu.VMEM((BK,), jnp.int32),
            pltpu.VMEM((BK,), jnp.int32),
            pltpu.VMEM((BK, ROW), jnp.float32),
            pltpu.SemaphoreType.DMA,
        ],
    )
    def body(t_ref, l_ref, f_ref, out_ref, lb, fb, idxv, rows, sem):
        w = lax.axis_index("s") * NC + lax.axis_index("c")

        def blk(i, carry):
            base = w * per_w + i * BK
            pltpu.sync_copy(l_ref.at[pl.ds(base, BK)], lb)
            pltpu.sync_copy(f_ref.at[pl.ds(base, BK)], fb)
            for j in range(BK // 16):
                s = pl.ds(j * 16, 16)
                idxv[s] = lb[s] * 3 + fb[s]
            pltpu.async_copy(t_ref.at[idxv], rows, sem).wait()
            pltpu.sync_copy(rows, out_ref.at[pl.ds(base, BK)])
            return carry

        lax.fori_loop(0, nblk, blk, 0)

    return body(t_hbm, lflat, fflat)


def kernel(letter_ids, fb_ids, turn_number, remaining, letter_table):
    b = letter_ids.shape[0]
    t = _build_table(letter_table)
    meta = _build_meta(turn_number, remaining)
    lflat = letter_ids.reshape(-1).astype(jnp.int32)
    fflat = fb_ids.reshape(-1).astype(jnp.int32)
    grid = jnp.take(t, lflat * 3 + fflat, axis=0)
    return grid.reshape(b, 6, 5, ROW), meta


# speedup vs baseline: 1.8222x; 1.8222x over previous
import functools

import jax
import jax.numpy as jnp
from jax import lax
from jax.experimental import pallas as pl
from jax.experimental.pallas import tpu as pltpu
from jax.experimental.pallas import tpu_sc as plsc

EMBED = 128
NFB = 3
NPOS = 5
ROW = EMBED + NFB

NC = 2
NS = 16
NW = NC * NS
S = 96


def _meta_body(t_ref, r_ref, o_ref):
    lane = lax.broadcasted_iota(jnp.int32, o_ref.shape, 1)
    o_ref[...] = jnp.where(lane == 0, t_ref[...], r_ref[...])


def _build_meta(turn_number, remaining):
    b = turn_number.shape[0]
    blk = 2048
    return pl.pallas_call(
        _meta_body,
        grid=(b // blk,),
        in_specs=[
            pl.BlockSpec((blk, 1), lambda i: (i, 0)),
            pl.BlockSpec((blk, 1), lambda i: (i, 0)),
        ],
        out_specs=pl.BlockSpec((blk, 2), lambda i: (i, 0)),
        out_shape=jax.ShapeDtypeStruct((b, 2), jnp.float32),
    )(turn_number.reshape(b, 1), remaining.reshape(b, 1))


def _sc_encode(letter_table, lflat, fflat, nslab):
    assert nslab % (NW * S) == 0
    per_w = nslab // NW
    nblk = per_w // S
    mesh = plsc.VectorSubcoreMesh(core_axis_name="c", subcore_axis_name="s")

    @functools.partial(
        pl.kernel,
        mesh=mesh,
        out_type=jax.ShapeDtypeStruct((nslab, NPOS, ROW), jnp.float32),
        compiler_params=pltpu.CompilerParams(needs_layout_passes=False),
        scratch_types=[
            pltpu.VMEM((S * NPOS,), jnp.int32),
            pltpu.VMEM((S * NPOS,), jnp.int32),
            pltpu.VMEM((S,), jnp.int32),
            pltpu.VMEM((S, 1, EMBED), jnp.float32),
            pltpu.VMEM((S, 1, NFB), jnp.float32),
            pltpu.SemaphoreType.DMA,
        ],
    )
    def body(tab_ref, l_ref, f_ref, out_ref, lb, fc, idxr, rows, fbb, sem):
        w = lax.axis_index("s") * NC + lax.axis_index("c")
        iota16 = lax.iota(jnp.int32, 16)
        ones = jnp.full((16,), 1.0, jnp.float32)
        zeros = jnp.zeros((16,), jnp.float32)

        def blk(i, carry):
            sbase = w * per_w + i * S
            pltpu.sync_copy(l_ref.at[pl.ds(sbase * NPOS, S * NPOS)], lb)
            pltpu.sync_copy(f_ref.at[pl.ds(sbase * NPOS, S * NPOS)], fc)
            for r in range(NPOS):
                for g in range(S // 16):
                    srow = g * 16 + iota16
                    flat = srow * NPOS + r
                    idxr[pl.ds(g * 16, 16)] = plsc.load_gather(lb, [flat])
                    fv = plsc.load_gather(fc, [flat])
                    zsplat = jnp.zeros((16,), jnp.int32)
                    for j in range(NFB):
                        plsc.store_scatter(
                            fbb, [srow, zsplat, jnp.full((16,), j, jnp.int32)], zeros)
                    plsc.store_scatter(fbb, [srow, zsplat, fv], ones)
                pltpu.async_copy(tab_ref.at[idxr], rows.at[:, 0], sem).wait()
                pltpu.sync_copy(
                    rows, out_ref.at[pl.ds(sbase, S), pl.ds(r, 1), pl.ds(0, EMBED)])
                pltpu.sync_copy(
                    fbb, out_ref.at[pl.ds(sbase, S), pl.ds(r, 1), pl.ds(EMBED, NFB)])
            return carry

        lax.fori_loop(0, nblk, blk, 0)

    return body(letter_table, lflat, fflat)


def kernel(letter_ids, fb_ids, turn_number, remaining, letter_table):
    b = letter_ids.shape[0]
    nslab = b * 6
    meta = _build_meta(turn_number, remaining)
    lflat = letter_ids.reshape(-1).astype(jnp.int32)
    fflat = fb_ids.reshape(-1).astype(jnp.int32)
    grid = _sc_encode(letter_table, lflat, fflat, nslab)
    return grid.reshape(b, 6, NPOS, ROW), meta


# speedup vs baseline: 1.8398x; 1.0097x over previous
import functools

import jax
import jax.numpy as jnp
from jax import lax
from jax.experimental import pallas as pl
from jax.experimental.pallas import tpu as pltpu
from jax.experimental.pallas import tpu_sc as plsc

EMBED = 128
NFB = 3
NPOS = 5
ROW = EMBED + NFB

NC = 2
NS = 16
NW = NC * NS
S = 64
NSLOT = 2


def _meta_body(t_ref, r_ref, o_ref):
    lane = lax.broadcasted_iota(jnp.int32, o_ref.shape, 1)
    o_ref[...] = jnp.where(lane == 0, t_ref[...], r_ref[...])


def _build_meta(turn_number, remaining):
    b = turn_number.shape[0]
    blk = 2048
    return pl.pallas_call(
        _meta_body,
        grid=(b // blk,),
        in_specs=[
            pl.BlockSpec((blk, 1), lambda i: (i, 0)),
            pl.BlockSpec((blk, 1), lambda i: (i, 0)),
        ],
        out_specs=pl.BlockSpec((blk, 2), lambda i: (i, 0)),
        out_shape=jax.ShapeDtypeStruct((b, 2), jnp.float32),
    )(turn_number.reshape(b, 1), remaining.reshape(b, 1))


def _slot_scratch():
    return (
        [pltpu.VMEM((S * NPOS,), jnp.int32)] * 2
        + [pltpu.VMEM((S,), jnp.int32)] * NPOS
        + [pltpu.VMEM((S, 1, EMBED), jnp.float32)] * NPOS
        + [pltpu.SemaphoreType.DMA]
        + [pltpu.SemaphoreType.DMA] * NPOS
        + [pltpu.SemaphoreType.DMA]
    )


_SLOT_LEN = 2 + NPOS + NPOS + 1 + NPOS + 1


def _global_scratch():
    return (
        [pltpu.VMEM((S, 1, NFB), jnp.float32)] * NPOS
        + [pltpu.SemaphoreType.DMA]
    )


def _sc_encode(letter_table, lflat, fflat, nslab):
    assert nslab % (NW * S) == 0
    per_w = nslab // NW
    nblk = per_w // S
    mesh = plsc.VectorSubcoreMesh(core_axis_name="c", subcore_axis_name="s")

    @functools.partial(
        pl.kernel,
        mesh=mesh,
        out_type=jax.ShapeDtypeStruct((nslab, NPOS, ROW), jnp.float32),
        compiler_params=pltpu.CompilerParams(needs_layout_passes=False),
        scratch_types=_slot_scratch() * NSLOT + _global_scratch(),
    )
    def body(tab_ref, l_ref, f_ref, out_ref, *scr):
        slots = []
        for sl in range(NSLOT):
            part = scr[sl * _SLOT_LEN:(sl + 1) * _SLOT_LEN]
            slots.append({
                "lb": part[0], "fc": part[1],
                "idx": part[2:2 + NPOS],
                "rows": part[2 + NPOS:2 + 2 * NPOS],
                "sem_in": part[2 + 2 * NPOS],
                "gsem": part[3 + 2 * NPOS:3 + 3 * NPOS],
                "sem_out": part[3 + 3 * NPOS],
            })
        gpart = scr[NSLOT * _SLOT_LEN:]
        fbb = gpart[:NPOS]
        sem_fb = gpart[NPOS]

        w = lax.axis_index("s") * NC + lax.axis_index("c")
        wbase = w * per_w
        iota16 = lax.iota(jnp.int32, 16)
        ones = jnp.full((16,), 1.0, jnp.float32)
        zeros = jnp.zeros((16,), jnp.float32)

        def in_copies(i, st):
            q0 = (wbase + i * S) * NPOS
            return (
                pltpu.make_async_copy(l_ref.at[pl.ds(q0, S * NPOS)], st["lb"], st["sem_in"]),
                pltpu.make_async_copy(f_ref.at[pl.ds(q0, S * NPOS)], st["fc"], st["sem_in"]),
            )

        def out_copies(i, st):
            sb = wbase + i * S
            return [
                pltpu.make_async_copy(
                    st["rows"][r],
                    out_ref.at[pl.ds(sb, S), pl.ds(r, 1), pl.ds(0, EMBED)],
                    st["sem_out"])
                for r in range(NPOS)
            ]

        def fb_copies(i):
            sb = wbase + i * S
            return [
                pltpu.make_async_copy(
                    fbb[r],
                    out_ref.at[pl.ds(sb, S), pl.ds(r, 1), pl.ds(EMBED, NFB)],
                    sem_fb)
                for r in range(NPOS)
            ]

        def fire_in(i, st):
            for c in in_copies(i, st):
                c.start()

        def step(i, carry):
            sl = lax.rem(i, 2)

            def run(st, ost):
                @pl.when(i >= NSLOT)
                def _():
                    for c in out_copies(i - NSLOT, st):
                        c.wait()
                for c in in_copies(i, st):
                    c.wait()
                for r in range(NPOS):
                    for g in range(S // 16):
                        sloc = g * 16 + iota16
                        st["idx"][r][pl.ds(g * 16, 16)] = plsc.load_gather(
                            st["lb"], [sloc * NPOS + r])
                @pl.when(i + 1 < nblk)
                def _():
                    fire_in(i + 1, ost)
                for r in range(NPOS):
                    pltpu.async_copy(
                        tab_ref.at[st["idx"][r]], st["rows"][r].at[:, 0],
                        st["gsem"][r])
                @pl.when(i >= 1)
                def _():
                    for c in fb_copies(i - 1):
                        c.wait()
                zsplat = jnp.zeros((16,), jnp.int32)
                for r in range(NPOS):
                    for g in range(S // 16):
                        sloc = g * 16 + iota16
                        fv = plsc.load_gather(st["fc"], [sloc * NPOS + r])
                        for j in range(NFB):
                            plsc.store_scatter(
                                fbb[r],
                                [sloc, zsplat, jnp.full((16,), j, jnp.int32)],
                                zeros)
                        plsc.store_scatter(fbb[r], [sloc, zsplat, fv], ones)
                cps = out_copies(i, st)
                for r in range(NPOS):
                    pltpu.make_async_copy(
                        tab_ref.at[st["idx"][r]], st["rows"][r].at[:, 0],
                        st["gsem"][r]).wait()
                    cps[r].start()
                for c in fb_copies(i):
                    c.start()

            @pl.when(sl == 0)
            def _():
                run(slots[0], slots[1])

            @pl.when(sl == 1)
            def _():
                run(slots[1], slots[0])

            return carry

        fire_in(0, slots[0])
        lax.fori_loop(0, nblk, step, 0)
        for c in out_copies(nblk - 2, slots[(nblk - 2) % 2]):
            c.wait()
        for c in out_copies(nblk - 1, slots[(nblk - 1) % 2]):
            c.wait()
        for c in fb_copies(nblk - 1):
            c.wait()

    return body(letter_table, lflat, fflat)


def kernel(letter_ids, fb_ids, turn_number, remaining, letter_table):
    b = letter_ids.shape[0]
    nslab = b * 6
    meta = _build_meta(turn_number, remaining)
    lflat = letter_ids.reshape(-1).astype(jnp.int32)
    fflat = fb_ids.reshape(-1).astype(jnp.int32)
    grid = _sc_encode(letter_table, lflat, fflat, nslab)
    return grid.reshape(b, 6, NPOS, ROW), meta


# speedup vs baseline: 1.9086x; 1.0374x over previous
import functools

import jax
import jax.numpy as jnp
from jax import lax
from jax.experimental import pallas as pl
from jax.experimental.pallas import tpu as pltpu
from jax.experimental.pallas import tpu_sc as plsc

EMBED = 128
NFB = 3
NPOS = 5
ROW = EMBED + NFB

NC = 2
NS = 16
NW = NC * NS
S = 64
NSLOT = 2


def _meta_body(t_ref, r_ref, o_ref):
    lane = lax.broadcasted_iota(jnp.int32, o_ref.shape, 1)
    o_ref[...] = jnp.where(lane == 0, t_ref[...], r_ref[...])


def _build_meta(turn_number, remaining):
    b = turn_number.shape[0]
    blk = 2048
    return pl.pallas_call(
        _meta_body,
        grid=(b // blk,),
        in_specs=[
            pl.BlockSpec((blk, 1), lambda i: (i, 0)),
            pl.BlockSpec((blk, 1), lambda i: (i, 0)),
        ],
        out_specs=pl.BlockSpec((blk, 2), lambda i: (i, 0)),
        out_shape=jax.ShapeDtypeStruct((b, 2), jnp.float32),
    )(turn_number.reshape(b, 1), remaining.reshape(b, 1))


def _slot_scratch():
    return (
        [pltpu.VMEM((S * NPOS,), jnp.int32)]
        + [pltpu.VMEM((S,), jnp.int32)] * NPOS
        + [pltpu.VMEM((S, 1, EMBED), jnp.float32)] * NPOS
        + [pltpu.SemaphoreType.DMA]
        + [pltpu.SemaphoreType.DMA] * NPOS
        + [pltpu.SemaphoreType.DMA]
    )


_SLOT_LEN = 1 + NPOS + NPOS + 1 + NPOS + 1


def _global_scratch():
    return (
        [pltpu.VMEM((S, 1, NFB), jnp.float32)] * NPOS
        + [pltpu.SemaphoreType.DMA]
    )


def _sc_encode(letter_table, cflat, nslab):
    assert nslab % (NW * S) == 0
    per_w = nslab // NW
    nblk = per_w // S
    mesh = plsc.VectorSubcoreMesh(core_axis_name="c", subcore_axis_name="s")

    @functools.partial(
        pl.kernel,
        mesh=mesh,
        out_type=jax.ShapeDtypeStruct((nslab, NPOS, ROW), jnp.float32),
        compiler_params=pltpu.CompilerParams(needs_layout_passes=False),
        scratch_types=_slot_scratch() * NSLOT + _global_scratch(),
    )
    def body(tab_ref, c_ref, out_ref, *scr):
        slots = []
        for sl in range(NSLOT):
            part = scr[sl * _SLOT_LEN:(sl + 1) * _SLOT_LEN]
            slots.append({
                "cb": part[0],
                "idx": part[1:1 + NPOS],
                "rows": part[1 + NPOS:1 + 2 * NPOS],
                "sem_in": part[1 + 2 * NPOS],
                "gsem": part[2 + 2 * NPOS:2 + 3 * NPOS],
                "sem_out": part[2 + 3 * NPOS],
            })
        gpart = scr[NSLOT * _SLOT_LEN:]
        fbb = gpart[:NPOS]
        sem_fb = gpart[NPOS]

        w = lax.axis_index("s") * NC + lax.axis_index("c")
        wbase = w * per_w
        iota16 = lax.iota(jnp.int32, 16)
        ones = jnp.full((16,), 1.0, jnp.float32)
        zeros = jnp.zeros((16,), jnp.float32)

        def in_copies(i, st):
            q0 = (wbase + i * S) * NPOS
            return (
                pltpu.make_async_copy(
                    c_ref.at[pl.ds(q0, S * NPOS)], st["cb"], st["sem_in"]),
            )

        def out_copies(i, st):
            sb = wbase + i * S
            return [
                pltpu.make_async_copy(
                    st["rows"][r],
                    out_ref.at[pl.ds(sb, S), pl.ds(r, 1), pl.ds(0, EMBED)],
                    st["sem_out"])
                for r in range(NPOS)
            ]

        def fb_copies(i):
            sb = wbase + i * S
            return [
                pltpu.make_async_copy(
                    fbb[r],
                    out_ref.at[pl.ds(sb, S), pl.ds(r, 1), pl.ds(EMBED, NFB)],
                    sem_fb)
                for r in range(NPOS)
            ]

        def fire_in(i, st):
            for c in in_copies(i, st):
                c.start()

        def step(i, carry):
            sl = lax.rem(i, 2)

            def run(st, ost):
                @pl.when(i >= NSLOT)
                def _():
                    for c in out_copies(i - NSLOT, st):
                        c.wait()
                for c in in_copies(i, st):
                    c.wait()
                for r in range(NPOS):
                    for g in range(S // 16):
                        sloc = g * 16 + iota16
                        cv = plsc.load_gather(st["cb"], [sloc * NPOS + r])
                        st["idx"][r][pl.ds(g * 16, 16)] = cv // NFB
                @pl.when(i + 1 < nblk)
                def _():
                    fire_in(i + 1, ost)
                for r in range(NPOS):
                    pltpu.async_copy(
                        tab_ref.at[st["idx"][r]], st["rows"][r].at[:, 0],
                        st["gsem"][r])
                @pl.when(i >= 1)
                def _():
                    for c in fb_copies(i - 1):
                        c.wait()
                zsplat = jnp.zeros((16,), jnp.int32)
                for r in range(NPOS):
                    for g in range(S // 16):
                        sloc = g * 16 + iota16
                        fv = plsc.load_gather(st["cb"], [sloc * NPOS + r]) % NFB
                        for j in range(NFB):
                            plsc.store_scatter(
                                fbb[r],
                                [sloc, zsplat, jnp.full((16,), j, jnp.int32)],
                                zeros)
                        plsc.store_scatter(fbb[r], [sloc, zsplat, fv], ones)
                cps = out_copies(i, st)
                for r in range(NPOS):
                    pltpu.make_async_copy(
                        tab_ref.at[st["idx"][r]], st["rows"][r].at[:, 0],
                        st["gsem"][r]).wait()
                    cps[r].start()
                for c in fb_copies(i):
                    c.start()

            @pl.when(sl == 0)
            def _():
                run(slots[0], slots[1])

            @pl.when(sl == 1)
            def _():
                run(slots[1], slots[0])

            return carry

        fire_in(0, slots[0])
        lax.fori_loop(0, nblk, step, 0)
        for c in out_copies(nblk - 2, slots[(nblk - 2) % 2]):
            c.wait()
        for c in out_copies(nblk - 1, slots[(nblk - 1) % 2]):
            c.wait()
        for c in fb_copies(nblk - 1):
            c.wait()

    return body(letter_table, cflat)


def kernel(letter_ids, fb_ids, turn_number, remaining, letter_table):
    b = letter_ids.shape[0]
    nslab = b * 6
    meta = _build_meta(turn_number, remaining)
    cflat = (letter_ids.astype(jnp.int32) * NFB
             + fb_ids.astype(jnp.int32)).reshape(-1)
    grid = _sc_encode(letter_table, cflat, nslab)
    return grid.reshape(b, 6, NPOS, ROW), meta
